# Initial kernel scaffold; baseline (speedup 1.0000x reference)
import jax
import jax.numpy as jnp
from jax.experimental import pallas as pl


def kernel(pos, bonds, angles, dihedrals, bond_k, bond_eq, angle_k, angle_eq, proper_ks):
    raise NotImplementedError("write your pallas kernel here")


# speedup vs baseline: 20.2983x; 20.2983x over previous
import functools

import numpy as np
import jax
import jax.numpy as jnp
from jax import lax
from jax.experimental import pallas as pl
from jax.experimental.pallas import tpu as pltpu, tpu_sc as plsc

_N = 100000
_E = 1600000
_K = 6
_NW = 32
_EW = _E // _NW
_C = 2560
_NCHT = _E // _C
_GF = 128
_NJF = _C // _GF
_NG = _C // 16

_F32 = jnp.float32
_I32 = jnp.int32
_SINPI = float(np.sin(np.float32(np.pi)))


def _full(v, dtype=_I32):
    return jnp.full((16,), v, dtype)


def _rsqrt(x):
    i = lax.bitcast_convert_type(x, _I32)
    i = jnp.int32(0x5F3759DF) - (i >> 1)
    y = lax.bitcast_convert_type(i, _F32)
    for _ in range(3):
        y = y * (1.5 - 0.5 * x * y * y)
    return y


def _cos_poly(t):
    t2 = t * t
    c = jnp.float32(-1.0 / 3628800.0)
    c = c * t2 + jnp.float32(1.0 / 40320.0)
    c = c * t2 + jnp.float32(-1.0 / 720.0)
    c = c * t2 + jnp.float32(1.0 / 24.0)
    c = c * t2 + jnp.float32(-0.5)
    c = c * t2 + jnp.float32(1.0)
    return c


def _sc_body(pos4, b0, b1, a0, a1, a2, d0, d1, d2, d3,
             bk, beq, ak, aeq, ks6, out,
             idx0, idx1, idx2, idx3, rows0, rows1, rows2, rows3,
             pk, pq, ks_0, ks_1, ks_2, ks_3, ks_4, ks_5,
             accv, isem, psem, gsem):
    idxb = [idx0, idx1, idx2, idx3]
    rows = [rows0, rows1, rows2, rows3]
    ksb = [ks_0, ks_1, ks_2, ks_3, ks_4, ks_5]
    wid = lax.axis_index("s") * 2 + lax.axis_index("c")
    accv[...] = jnp.zeros((16,), _F32)

    def gat(s, comp, gi):
        return plsc.load_gather(rows[s], [gi, _full(comp)])

    def bond_grp(gi, g16):
        axc, ayc, azc = gat(0, 0, gi), gat(0, 1, gi), gat(0, 2, gi)
        bxc, byc, bzc = gat(1, 0, gi), gat(1, 1, gi), gat(1, 2, gi)
        dx, dy, dz = axc - bxc, ayc - byc, azc - bzc
        d2 = dx * dx + dy * dy + dz * dz + 1e-12
        d = d2 * _rsqrt(d2)
        k = pk[pl.ds(g16, 16)]
        eq = pq[pl.ds(g16, 16)]
        t = d - eq
        accv[...] = accv[...] + 0.5 * k * t * t

    def ang_grp(gi, g16):
        ix, iy, iz = gat(0, 0, gi), gat(0, 1, gi), gat(0, 2, gi)
        jx, jy, jz = gat(1, 0, gi), gat(1, 1, gi), gat(1, 2, gi)
        kx, ky, kz = gat(2, 0, gi), gat(2, 1, gi), gat(2, 2, gi)
        jix, jiy, jiz = ix - jx, iy - jy, iz - jz
        jkx, jky, jkz = kx - jx, ky - jy, kz - jz
        dot = jix * jkx + jiy * jky + jiz * jkz
        a = jix * jix + jiy * jiy + jiz * jiz + 1e-12
        b = jkx * jkx + jky * jky + jkz * jkz + 1e-12
        ab = a * b
        denom = ab * _rsqrt(ab) + 1e-9
        cosang = dot / denom
        k = pk[pl.ds(g16, 16)]
        eq = pq[pl.ds(g16, 16)]
        t = cosang - _cos_poly(eq)
        accv[...] = accv[...] + 0.5 * k * t * t

    def dih_grp(gi, g16):
        ix, iy, iz = gat(0, 0, gi), gat(0, 1, gi), gat(0, 2, gi)
        jx, jy, jz = gat(1, 0, gi), gat(1, 1, gi), gat(1, 2, gi)
        kx, ky, kz = gat(2, 0, gi), gat(2, 1, gi), gat(2, 2, gi)
        lx, ly, lz = gat(3, 0, gi), gat(3, 1, gi), gat(3, 2, gi)
        b1x, b1y, b1z = jx - ix, jy - iy, jz - iz
        b2x, b2y, b2z = kx - jx, ky - jy, kz - jz
        b3x, b3y, b3z = lx - kx, ly - ky, lz - kz
        n1x = b1y * b2z - b1z * b2y
        n1y = b1z * b2x - b1x * b2z
        n1z = b1x * b2y - b1y * b2x
        n2x = b2y * b3z - b2z * b3y
        n2y = b2z * b3x - b2x * b3z
        n2z = b2x * b3y - b2y * b3x
        b2sq = b2x * b2x + b2y * b2y + b2z * b2z + 1e-12
        rb2 = _rsqrt(b2sq)
        ux, uy, uz = b2x * rb2, b2y * rb2, b2z * rb2
        m1x = n1y * uz - n1z * uy
        m1y = n1z * ux - n1x * uz
        m1z = n1x * uy - n1y * ux
        x = n1x * n2x + n1y * n2y + n1z * n2z + 1e-9
        y = m1x * n2x + m1y * n2y + m1z * n2z
        r2 = jnp.maximum(x * x + y * y, jnp.float32(1e-30))
        ri = _rsqrt(r2)
        c1, s1 = x * ri, y * ri
        e = jnp.zeros((16,), _F32)
        cp, sp = jnp.ones((16,), _F32), jnp.zeros((16,), _F32)
        cc, sc = c1, s1
        two_c1 = c1 + c1
        for h in range(_K):
            ksh = ksb[h][pl.ds(g16, 16)]
            e = e + jnp.abs(ksh) + ksh * cc + \
                jnp.float32(_SINPI) * jnp.maximum(-ksh, 0.0) * sc
            cn = two_c1 * cc - cp
            sn = two_c1 * sc - sp
            cp, sp, cc, sc = cc, sc, cn, sn
        accv[...] = accv[...] + e

    nch_w = (jnp.int32(_NCHT) - wid + jnp.int32(_NW - 1)) // jnp.int32(_NW)

    def run_phase(nslots, idx_srcs, par_pairs, grp_fn):
        def chunk_body(ci, _):
            base = (wid + ci * _NW) * _C
            idescs = [
                pltpu.async_copy(src.at[pl.ds(base, _C)], idxb[s], isem)
                for s, src in enumerate(idx_srcs)
            ]
            pdescs = [
                pltpu.async_copy(src.at[pl.ds(base, _C)], dst, psem)
                for src, dst in par_pairs
            ]
            for dsc in idescs:
                dsc.wait()
            def fire(j, _):
                off = pl.multiple_of(j * _GF, _GF)
                for s in range(nslots):
                    pltpu.async_copy(
                        pos4.at[idxb[s].at[pl.ds(off, _GF)]],
                        rows[s].at[pl.ds(off, _GF)], gsem)
                return 0
            lax.fori_loop(0, _NJF, fire, 0)
            for dsc in pdescs:
                dsc.wait()
            def drain(j, _):
                off = pl.multiple_of(j * _GF, _GF)
                for s in range(nslots):
                    pltpu.make_async_copy(
                        pos4.at[idxb[s].at[pl.ds(off, _GF)]],
                        rows[s].at[pl.ds(off, _GF)], gsem).wait()
                return 0
            lax.fori_loop(0, _NJF, drain, 0)
            def grp(g, _):
                g16 = pl.multiple_of(g * 16, 16)
                gi = g16 + lax.iota(_I32, 16)
                grp_fn(gi, g16)
                return 0
            lax.fori_loop(0, _NG, grp, 0)
            return 0
        lax.fori_loop(0, nch_w, chunk_body, 0)

    run_phase(2, [b0, b1], [(bk, pk), (beq, pq)], bond_grp)
    run_phase(3, [a0, a1, a2], [(ak, pk), (aeq, pq)], ang_grp)
    run_phase(4, [d0, d1, d2, d3],
              [(ks6.at[h], ksb[h]) for h in range(_K)], dih_grp)

    pltpu.sync_copy(accv, out.at[wid])


@functools.lru_cache(maxsize=1)
def _build():
    mesh = plsc.VectorSubcoreMesh(core_axis_name="c", subcore_axis_name="s")
    return pl.kernel(
        _sc_body,
        out_type=jax.ShapeDtypeStruct((_NW, 16), _F32),
        mesh=mesh,
        compiler_params=pltpu.CompilerParams(needs_layout_passes=False, use_tc_tiling_on_sc=False),
        scratch_types=[
            pltpu.VMEM((_C,), _I32),
            pltpu.VMEM((_C,), _I32),
            pltpu.VMEM((_C,), _I32),
            pltpu.VMEM((_C,), _I32),
            pltpu.VMEM((_C, 4), _F32),
            pltpu.VMEM((_C, 4), _F32),
            pltpu.VMEM((_C, 4), _F32),
            pltpu.VMEM((_C, 4), _F32),
            pltpu.VMEM((_C,), _F32),
            pltpu.VMEM((_C,), _F32),
            pltpu.VMEM((_C,), _F32),
            pltpu.VMEM((_C,), _F32),
            pltpu.VMEM((_C,), _F32),
            pltpu.VMEM((_C,), _F32),
            pltpu.VMEM((_C,), _F32),
            pltpu.VMEM((_C,), _F32),
            pltpu.VMEM((16,), _F32),
            pltpu.SemaphoreType.DMA,
            pltpu.SemaphoreType.DMA,
            pltpu.SemaphoreType.DMA,
        ],
    )


def kernel(pos, bonds, angles, dihedrals, bond_k, bond_eq,
           angle_k, angle_eq, proper_ks):
    pos = pos.astype(_F32)
    pos4 = jnp.concatenate([pos, jnp.zeros((_N, 1), _F32)], axis=1)
    b = bonds.astype(_I32)
    a = angles.astype(_I32)
    dh = dihedrals.astype(_I32)
    parts = _build()(
        pos4, b[0], b[1], a[0], a[1], a[2], dh[0], dh[1], dh[2], dh[3],
        bond_k.astype(_F32), bond_eq.astype(_F32),
        angle_k.astype(_F32), angle_eq.astype(_F32),
        proper_ks.astype(_F32).T)
    return jnp.sum(parts)
